# Initial kernel scaffold; baseline (speedup 1.0000x reference)
import jax
import jax.numpy as jnp
from jax.experimental import pallas as pl


def kernel(layer_attention_probes, mask):
    raise NotImplementedError("write your pallas kernel here")


# speedup vs baseline: 1.1340x; 1.1340x over previous
import functools

import jax
import jax.numpy as jnp
import numpy as np
from jax import lax
from jax.experimental import pallas as pl
from jax.experimental.pallas import tpu as pltpu
from jax.experimental.pallas import tpu_sc as plsc

PRUNE_KEEP = 1.0 - 0.6
LANES = 16
INT32_MIN = np.int32(-2147483648)


def _reduce_body(x_ref, o_ref, *, denom):
    i = pl.program_id(1)

    @pl.when(i == 0)
    def _init():
        o_ref[...] = jnp.zeros_like(o_ref)

    o_ref[...] += jnp.sum(x_ref[...], axis=1, keepdims=True)

    @pl.when(i == pl.num_programs(1) - 1)
    def _scale():
        o_ref[...] = o_ref[...] / jnp.float32(denom)


def _mean_scores(probes):
    bs, r_total, s = probes.shape
    block_r = 512
    grid = (bs, r_total // block_r)
    out = pl.pallas_call(
        functools.partial(_reduce_body, denom=r_total),
        grid=grid,
        in_specs=[pl.BlockSpec((1, block_r, s), lambda b, i: (b, i, 0))],
        out_specs=pl.BlockSpec((1, 1, s), lambda b, i: (b, 0, 0)),
        out_shape=jax.ShapeDtypeStruct((bs, 1, s), jnp.float32),
    )(probes)
    return out.reshape(bs, s)


def _perm(v, idx):
    return v.at[idx].get(mode="promise_in_bounds")


def _vreduce_sum(v):
    iota = lax.iota(jnp.int32, LANES)
    for sft in (8, 4, 2, 1):
        v = v + _perm(v, iota ^ sft)
    return v


def _vcumsum(v):
    iota = lax.iota(jnp.int32, LANES)
    zero = jnp.zeros_like(v)
    for sft in (1, 2, 4, 8):
        g = _perm(v, jnp.maximum(iota - sft, 0))
        v = v + jnp.where(iota >= sft, g, zero)
    return v


def _select_body(scores_hbm, mask_hbm, out_hbm, scores_v, mask_v, keys_v,
                 out_v, *, bs, s):
    chunks = s // LANES
    nc = 2
    cid = lax.axis_index("c")
    sid = lax.axis_index("s")
    wid = sid * nc + cid

    @pl.when(wid < bs)
    def _row():
        b = wid
        pltpu.sync_copy(scores_hbm.at[b], scores_v)
        pltpu.sync_copy(mask_hbm.at[b], mask_v)

        ones_i = jnp.ones((LANES,), jnp.int32)
        zeros_i = jnp.zeros((LANES,), jnp.int32)

        def _msum(i, acc):
            m = mask_v[pl.ds(i * LANES, LANES)]
            return acc + jnp.where(m != 0.0, ones_i, zeros_i)

        total_i = _vreduce_sum(lax.fori_loop(0, chunks, _msum, zeros_i))
        sep = total_i - 1
        k = (total_i.astype(jnp.float32) *
             jnp.float32(PRUNE_KEEP)).astype(jnp.int32)

        def _mkkeys(i, carry):
            x = scores_v[pl.ds(i * LANES, LANES)]
            u = lax.bitcast_convert_type(x, jnp.int32)
            sgn = lax.shift_right_arithmetic(u, 31)
            key = u ^ (sgn & jnp.int32(0x7FFFFFFF))
            gidx = i * LANES + lax.iota(jnp.int32, LANES)
            excl = (gidx == 0) | (gidx == sep)
            keys_v[pl.ds(i * LANES, LANES)] = jnp.where(
                excl, jnp.full((LANES,), INT32_MIN, jnp.int32), key)
            return carry

        lax.fori_loop(0, chunks, _mkkeys, jnp.int32(0))

        def _count_ge(cand_s):
            def _c(i, acc):
                kc = keys_v[pl.ds(i * LANES, LANES)]
                return acc + jnp.where(kc >= cand_s, ones_i, zeros_i)

            return _vreduce_sum(lax.fori_loop(0, chunks, _c, zeros_i))

        def _bit(t, p):
            bit = lax.shift_left(jnp.int32(1), 31 - t)
            cand = p | bit
            cnt = _count_ge(cand ^ INT32_MIN)
            return jnp.where(cnt >= k, cand, p)

        p_final = lax.fori_loop(0, 32, _bit, zeros_i)
        t_s = p_final ^ INT32_MIN

        def _cgt(i, accs):
            a_gt, a_valid = accs
            kc = keys_v[pl.ds(i * LANES, LANES)]
            return (a_gt + jnp.where(kc > t_s, ones_i, zeros_i),
                    a_valid + jnp.where(kc > INT32_MIN, ones_i, zeros_i))

        acc_gt, acc_valid = lax.fori_loop(0, chunks, _cgt, (zeros_i, zeros_i))
        cnt_gt = _vreduce_sum(acc_gt)
        valid_cnt = _vreduce_sum(acc_valid)
        need = k - cnt_gt
        fallback = (valid_cnt == 0) | (k <= 0)

        def _emit(i, carry):
            kc = keys_v[pl.ds(i * LANES, LANES)]
            eq = kc == t_s
            eqi = jnp.where(eq, ones_i, zeros_i)
            rank = carry + _vcumsum(eqi)
            tie = eq & (rank <= need)
            gidx = i * LANES + lax.iota(jnp.int32, LANES)
            force = (gidx == 0) | (gidx == sep)
            sel = (kc > t_s) | tie | force
            row = jnp.where(sel, jnp.ones((LANES,), jnp.float32),
                            jnp.zeros((LANES,), jnp.float32))
            mrow = mask_v[pl.ds(i * LANES, LANES)]
            out_v[pl.ds(i * LANES, LANES)] = jnp.where(fallback, mrow, row)
            return carry + _vreduce_sum(eqi)

        lax.fori_loop(0, chunks, _emit, zeros_i)
        pltpu.sync_copy(out_v, out_hbm.at[b])


def _select_sc(scores, mask2):
    bs, s = scores.shape
    mesh = plsc.VectorSubcoreMesh(core_axis_name="c", subcore_axis_name="s")
    kern = functools.partial(
        pl.kernel,
        mesh=mesh,
        out_type=jax.ShapeDtypeStruct((bs, s), jnp.float32),
        scratch_types=[
            pltpu.VMEM((s,), jnp.float32),
            pltpu.VMEM((s,), jnp.float32),
            pltpu.VMEM((s,), jnp.int32),
            pltpu.VMEM((s,), jnp.float32),
        ],
    )(functools.partial(_select_body, bs=bs, s=s))
    return kern(scores, mask2)


def kernel(layer_attention_probes, mask):
    bs, h, s_rows, s = layer_attention_probes.shape
    probes = layer_attention_probes.reshape(bs, h * s_rows, s)
    scores = _mean_scores(probes)
    mask2 = mask.reshape(bs, s)
    return _select_sc(scores, mask2)


# speedup vs baseline: 1.1848x; 1.0448x over previous
import functools

import jax
import jax.numpy as jnp
import numpy as np
from jax import lax
from jax.experimental import pallas as pl
from jax.experimental.pallas import tpu as pltpu
from jax.experimental.pallas import tpu_sc as plsc

PRUNE_KEEP = 1.0 - 0.6
LANES = 16
INT32_MIN = np.int32(-2147483648)


def _reduce_body(x_ref, o_ref, *, denom):
    i = pl.program_id(1)

    @pl.when(i == 0)
    def _init():
        o_ref[...] = jnp.zeros_like(o_ref)

    o_ref[...] += jnp.sum(x_ref[...], axis=1, keepdims=True)

    @pl.when(i == pl.num_programs(1) - 1)
    def _scale():
        o_ref[...] = o_ref[...] / jnp.float32(denom)


def _mean_scores(probes):
    bs, r_total, s = probes.shape
    block_r = 1024
    grid = (bs, r_total // block_r)
    out = pl.pallas_call(
        functools.partial(_reduce_body, denom=r_total),
        grid=grid,
        in_specs=[pl.BlockSpec((1, block_r, s), lambda b, i: (b, i, 0))],
        out_specs=pl.BlockSpec((1, 1, s), lambda b, i: (b, 0, 0)),
        out_shape=jax.ShapeDtypeStruct((bs, 1, s), jnp.float32),
    )(probes)
    return out.reshape(bs, s)


def _perm(v, idx):
    return v.at[idx].get(mode="promise_in_bounds")


def _vreduce_sum(v):
    iota = lax.iota(jnp.int32, LANES)
    for sft in (8, 4, 2, 1):
        v = v + _perm(v, iota ^ sft)
    return v


def _vcumsum(v):
    iota = lax.iota(jnp.int32, LANES)
    zero = jnp.zeros_like(v)
    for sft in (1, 2, 4, 8):
        g = _perm(v, jnp.maximum(iota - sft, 0))
        v = v + jnp.where(iota >= sft, g, zero)
    return v


def _select_body(scores_hbm, mask_hbm, out_hbm, scores_v, mask_v, keys_v,
                 out_v, *, bs, s):
    chunks = s // LANES
    nc = 2
    cid = lax.axis_index("c")
    sid = lax.axis_index("s")
    wid = sid * nc + cid

    @pl.when(wid < bs)
    def _row():
        b = wid
        pltpu.sync_copy(scores_hbm.at[b], scores_v)
        pltpu.sync_copy(mask_hbm.at[b], mask_v)

        ones_i = jnp.ones((LANES,), jnp.int32)
        zeros_i = jnp.zeros((LANES,), jnp.int32)

        def _msum(i, acc):
            m = mask_v[pl.ds(i * LANES, LANES)]
            return acc + jnp.where(m != 0.0, ones_i, zeros_i)

        total_i = _vreduce_sum(lax.fori_loop(0, chunks, _msum, zeros_i))
        sep = total_i - 1
        k = (total_i.astype(jnp.float32) *
             jnp.float32(PRUNE_KEEP)).astype(jnp.int32)

        def _mkkeys(i, carry):
            x = scores_v[pl.ds(i * LANES, LANES)]
            u = lax.bitcast_convert_type(x, jnp.int32)
            sgn = lax.shift_right_arithmetic(u, 31)
            key = u ^ (sgn & jnp.int32(0x7FFFFFFF))
            gidx = i * LANES + lax.iota(jnp.int32, LANES)
            excl = (gidx == 0) | (gidx == sep)
            keys_v[pl.ds(i * LANES, LANES)] = jnp.where(
                excl, jnp.full((LANES,), INT32_MIN, jnp.int32), key)
            return carry

        lax.fori_loop(0, chunks, _mkkeys, jnp.int32(0))

        def _count_ge(cand_s):
            def _c(i, acc):
                kc = keys_v[pl.ds(i * LANES, LANES)]
                return acc + jnp.where(kc >= cand_s, ones_i, zeros_i)

            return _vreduce_sum(lax.fori_loop(0, chunks, _c, zeros_i))

        def _bit(t, p):
            bit = lax.shift_left(jnp.int32(1), 31 - t)
            cand = p | bit
            cnt = _count_ge(cand ^ INT32_MIN)
            return jnp.where(cnt >= k, cand, p)

        p_final = lax.fori_loop(0, 32, _bit, zeros_i)
        t_s = p_final ^ INT32_MIN

        def _cgt(i, accs):
            a_gt, a_valid = accs
            kc = keys_v[pl.ds(i * LANES, LANES)]
            return (a_gt + jnp.where(kc > t_s, ones_i, zeros_i),
                    a_valid + jnp.where(kc > INT32_MIN, ones_i, zeros_i))

        acc_gt, acc_valid = lax.fori_loop(0, chunks, _cgt, (zeros_i, zeros_i))
        cnt_gt = _vreduce_sum(acc_gt)
        valid_cnt = _vreduce_sum(acc_valid)
        need = k - cnt_gt
        fallback = (valid_cnt == 0) | (k <= 0)

        def _emit(i, carry):
            kc = keys_v[pl.ds(i * LANES, LANES)]
            eq = kc == t_s
            eqi = jnp.where(eq, ones_i, zeros_i)
            rank = carry + _vcumsum(eqi)
            tie = eq & (rank <= need)
            gidx = i * LANES + lax.iota(jnp.int32, LANES)
            force = (gidx == 0) | (gidx == sep)
            sel = (kc > t_s) | tie | force
            row = jnp.where(sel, jnp.ones((LANES,), jnp.float32),
                            jnp.zeros((LANES,), jnp.float32))
            mrow = mask_v[pl.ds(i * LANES, LANES)]
            out_v[pl.ds(i * LANES, LANES)] = jnp.where(fallback, mrow, row)
            return carry + _vreduce_sum(eqi)

        lax.fori_loop(0, chunks, _emit, zeros_i)
        pltpu.sync_copy(out_v, out_hbm.at[b])


def _select_sc(scores, mask2):
    bs, s = scores.shape
    mesh = plsc.VectorSubcoreMesh(core_axis_name="c", subcore_axis_name="s")
    kern = functools.partial(
        pl.kernel,
        mesh=mesh,
        out_type=jax.ShapeDtypeStruct((bs, s), jnp.float32),
        scratch_types=[
            pltpu.VMEM((s,), jnp.float32),
            pltpu.VMEM((s,), jnp.float32),
            pltpu.VMEM((s,), jnp.int32),
            pltpu.VMEM((s,), jnp.float32),
        ],
    )(functools.partial(_select_body, bs=bs, s=s))
    return kern(scores, mask2)


def kernel(layer_attention_probes, mask):
    bs, h, s_rows, s = layer_attention_probes.shape
    probes = layer_attention_probes.reshape(bs, h * s_rows, s)
    scores = _mean_scores(probes)
    mask2 = mask.reshape(bs, s)
    return _select_sc(scores, mask2)
